# baseline (speedup 1.0000x reference)
import functools

import jax
import jax.numpy as jnp
from jax import lax
from jax.experimental import pallas as pl
from jax.experimental.pallas import tpu as pltpu
from jax.experimental.pallas import tpu_sc as plsc

BEAM = 64
BATCH = 64
PRE = 96
VOCAB = 1000000

_NC = 2
_NS = 16
_NW = _NC * _NS
_PER_W = (BATCH * PRE) // _NW
_ROWS_W = _PER_W // PRE
_SEG = 128
_NV = PRE // 16


def _sc_beam(scores, part_flat):
    mesh = plsc.VectorSubcoreMesh(core_axis_name="c", subcore_axis_name="s")
    out_types = [
        jax.ShapeDtypeStruct((BATCH * BEAM,), jnp.float32),
        jax.ShapeDtypeStruct((BATCH * BEAM,), jnp.int32),
        jax.ShapeDtypeStruct((BATCH * BEAM,), jnp.float32),
        jax.ShapeDtypeStruct((BATCH * BEAM,), jnp.int32),
    ]

    @functools.partial(
        pl.kernel,
        mesh=mesh,
        out_type=out_types,
        scratch_types=[
            pltpu.VMEM((_PER_W,), jnp.int32),
            pltpu.VMEM((16, 8, _SEG), jnp.float32),
            pltpu.VMEM((_PER_W,), jnp.float32),
            pltpu.VMEM((_ROWS_W * BEAM,), jnp.float32),
            pltpu.VMEM((_ROWS_W * BEAM,), jnp.int32),
            pltpu.VMEM((_ROWS_W * BEAM,), jnp.float32),
            pltpu.VMEM((_ROWS_W * BEAM,), jnp.int32),
            pltpu.SemaphoreType.DMA,
        ],
        compiler_params=pltpu.CompilerParams(use_tc_tiling_on_sc=True,
                                             needs_layout_passes=False),
    )
    def body(scores_hbm, part_hbm, tv_hbm, ti_hbm, lv_hbm, li_hbm,
             part_v, bufs, vals_v, tv_v, ti_v, lv_v, li_v, sem):
        wid = lax.axis_index("s") * _NC + lax.axis_index("c")
        base = wid * _PER_W
        pltpu.sync_copy(part_hbm.at[pl.ds(base, _PER_W)], part_v)
        iota16 = lax.iota(jnp.int32, 16)
        neg = jnp.float32(-jnp.inf)
        big = jnp.int32(2**30)

        for h in range(_ROWS_W):
            row = wid * _ROWS_W + h
            row8 = pl.multiple_of((row // 8) * 8, 8)
            subl = jnp.full((16,), row & 7, jnp.int32)
            for g in range(PRE // 16):
                off = h * PRE + g * 16
                chunk = part_v[pl.ds(off, 16)]
                segs = (chunk // _SEG) * _SEG
                copies = []
                for i in range(16):
                    col0 = pl.multiple_of(segs[i], _SEG)
                    copies.append(pltpu.async_copy(
                        scores_hbm.at[pl.ds(row8, 8), pl.ds(col0, _SEG)],
                        bufs.at[i], sem))
                for cp in copies:
                    cp.wait()
                lanes = chunk & (_SEG - 1)
                vals_v[pl.ds(off, 16)] = plsc.load_gather(
                    bufs, [iota16, subl, lanes])

        for h in range(_ROWS_W):
            off = h * PRE
            gv = [vals_v[pl.ds(off + 16 * c, 16)] for c in range(_NV)]
            idv = [part_v[pl.ds(off + 16 * c, 16)] for c in range(_NV)]
            pos = [iota16 + 16 * c for c in range(_NV)]
            lvv = list(gv)
            for b in range(BEAM // 16):

                def step(i, carry):
                    g = list(carry[0:_NV])
                    l = list(carry[_NV:2 * _NV])
                    atv, ati, alv, ali = carry[2 * _NV:]
                    lane = iota16 == i
                    m = g[0]
                    for c in range(1, _NV):
                        m = jnp.maximum(m, g[c])
                    mx = jnp.max(m)
                    cidv = jnp.where(g[0] == mx, idv[0], big)
                    for c in range(1, _NV):
                        cidv = jnp.minimum(cidv,
                                           jnp.where(g[c] == mx, idv[c], big))
                    cid = jnp.min(cidv)
                    atv = jnp.where(lane, mx, atv)
                    ati = jnp.where(lane, cid, ati)
                    g = [jnp.where(idv[c] == cid, neg, g[c])
                         for c in range(_NV)]
                    ml = l[0]
                    for c in range(1, _NV):
                        ml = jnp.maximum(ml, l[c])
                    mlx = jnp.max(ml)
                    cjv = jnp.where(l[0] == mlx, pos[0], big)
                    for c in range(1, _NV):
                        cjv = jnp.minimum(cjv,
                                          jnp.where(l[c] == mlx, pos[c], big))
                    cj = jnp.min(cjv)
                    alv = jnp.where(lane, mlx, alv)
                    ali = jnp.where(lane, cj, ali)
                    l = [jnp.where(pos[c] == cj, neg, l[c])
                         for c in range(_NV)]
                    return (*g, *l, atv, ati, alv, ali)

                zf = jnp.zeros((16,), jnp.float32)
                zi = jnp.zeros((16,), jnp.int32)
                carry = lax.fori_loop(0, 16, step, (*gv, *lvv, zf, zi, zf, zi))
                gv = list(carry[0:_NV])
                lvv = list(carry[_NV:2 * _NV])
                atv, ati, alv, ali = carry[2 * _NV:]
                o = h * BEAM + b * 16
                tv_v[pl.ds(o, 16)] = atv
                ti_v[pl.ds(o, 16)] = ati
                lv_v[pl.ds(o, 16)] = alv
                li_v[pl.ds(o, 16)] = ali

        obase = wid * (_ROWS_W * BEAM)
        pltpu.sync_copy(tv_v, tv_hbm.at[pl.ds(obase, _ROWS_W * BEAM)])
        pltpu.sync_copy(ti_v, ti_hbm.at[pl.ds(obase, _ROWS_W * BEAM)])
        pltpu.sync_copy(lv_v, lv_hbm.at[pl.ds(obase, _ROWS_W * BEAM)])
        pltpu.sync_copy(li_v, li_hbm.at[pl.ds(obase, _ROWS_W * BEAM)])

    return body(scores, part_flat)


def kernel(weighted_scores, part_ids):
    tv, ti, lv, li = _sc_beam(weighted_scores, part_ids.reshape(-1))
    return (tv.reshape(BATCH, BEAM), ti.reshape(BATCH, BEAM),
            lv.reshape(BATCH, BEAM), li.reshape(BATCH, BEAM))


# speedup vs baseline: 1.0951x; 1.0951x over previous
import functools

import jax
import jax.numpy as jnp
from jax import lax
from jax.experimental import pallas as pl
from jax.experimental.pallas import tpu as pltpu
from jax.experimental.pallas import tpu_sc as plsc

BEAM = 64
BATCH = 64
PRE = 96
VOCAB = 1000000

_NC = 2
_NS = 16
_NW = _NC * _NS
_PER_W = (BATCH * PRE) // _NW
_ROWS_W = _PER_W // PRE
_SEG = 128
_NV = PRE // 16


def _sc_beam(scores, part_flat):
    mesh = plsc.VectorSubcoreMesh(core_axis_name="c", subcore_axis_name="s")
    out_types = [
        jax.ShapeDtypeStruct((BATCH * BEAM,), jnp.float32),
        jax.ShapeDtypeStruct((BATCH * BEAM,), jnp.int32),
        jax.ShapeDtypeStruct((BATCH * BEAM,), jnp.float32),
        jax.ShapeDtypeStruct((BATCH * BEAM,), jnp.int32),
    ]

    @functools.partial(
        pl.kernel,
        mesh=mesh,
        out_type=out_types,
        scratch_types=[
            pltpu.VMEM((_PER_W,), jnp.int32),
            pltpu.VMEM((2, 16, 8, _SEG), jnp.float32),
            pltpu.VMEM((_PER_W,), jnp.float32),
            pltpu.VMEM((_ROWS_W * BEAM,), jnp.float32),
            pltpu.VMEM((_ROWS_W * BEAM,), jnp.int32),
            pltpu.VMEM((_ROWS_W * BEAM,), jnp.float32),
            pltpu.VMEM((_ROWS_W * BEAM,), jnp.int32),
            pltpu.SemaphoreType.DMA,
            pltpu.SemaphoreType.DMA,
        ],
        compiler_params=pltpu.CompilerParams(use_tc_tiling_on_sc=True,
                                             needs_layout_passes=False),
    )
    def body(scores_hbm, part_hbm, tv_hbm, ti_hbm, lv_hbm, li_hbm,
             part_v, bufs, vals_v, tv_v, ti_v, lv_v, li_v, sem_a, sem_b):
        wid = lax.axis_index("s") * _NC + lax.axis_index("c")
        base = wid * _PER_W
        pltpu.sync_copy(part_hbm.at[pl.ds(base, _PER_W)], part_v)
        iota16 = lax.iota(jnp.int32, 16)
        neg = jnp.float32(-jnp.inf)
        big = jnp.int32(2**30)
        sems = (sem_a, sem_b)

        n_groups = _ROWS_W * (PRE // 16)

        def issue(gl):
            h, g = divmod(gl, PRE // 16)
            row = wid * _ROWS_W + h
            row8 = pl.multiple_of((row // 8) * 8, 8)
            off = h * PRE + g * 16
            chunk = part_v[pl.ds(off, 16)]
            segs = (chunk // _SEG) * _SEG
            copies = []
            for i in range(16):
                col0 = pl.multiple_of(segs[i], _SEG)
                copies.append(pltpu.async_copy(
                    scores_hbm.at[pl.ds(row8, 8), pl.ds(col0, _SEG)],
                    bufs.at[gl % 2, i], sems[gl % 2]))
            return copies, chunk, off, row

        def drain(state):
            copies, chunk, off, row = state
            for cp in copies:
                cp.wait()
            subl = jnp.full((16,), row & 7, jnp.int32)
            lanes = chunk & (_SEG - 1)
            vals_v[pl.ds(off, 16)] = plsc.load_gather(
                bufs.at[(off // 16) % 2], [iota16, subl, lanes])

        prev = issue(0)
        for gl in range(1, n_groups):
            cur = issue(gl)
            drain(prev)
            prev = cur
        drain(prev)

        for h in range(_ROWS_W):
            off = h * PRE
            gv = [vals_v[pl.ds(off + 16 * c, 16)] for c in range(_NV)]
            idv = [part_v[pl.ds(off + 16 * c, 16)] for c in range(_NV)]
            pos = [iota16 + 16 * c for c in range(_NV)]
            lvv = list(gv)
            for b in range(BEAM // 16):

                def step(i, carry):
                    g = list(carry[0:_NV])
                    l = list(carry[_NV:2 * _NV])
                    atv, ati, alv, ali = carry[2 * _NV:]
                    lane = iota16 == i
                    m = g[0]
                    for c in range(1, _NV):
                        m = jnp.maximum(m, g[c])
                    mx = jnp.max(m)
                    cidv = jnp.where(g[0] == mx, idv[0], big)
                    for c in range(1, _NV):
                        cidv = jnp.minimum(cidv,
                                           jnp.where(g[c] == mx, idv[c], big))
                    cid = jnp.min(cidv)
                    atv = jnp.where(lane, mx, atv)
                    ati = jnp.where(lane, cid, ati)
                    g = [jnp.where(idv[c] == cid, neg, g[c])
                         for c in range(_NV)]
                    ml = l[0]
                    for c in range(1, _NV):
                        ml = jnp.maximum(ml, l[c])
                    mlx = jnp.max(ml)
                    cjv = jnp.where(l[0] == mlx, pos[0], big)
                    for c in range(1, _NV):
                        cjv = jnp.minimum(cjv,
                                          jnp.where(l[c] == mlx, pos[c], big))
                    cj = jnp.min(cjv)
                    alv = jnp.where(lane, mlx, alv)
                    ali = jnp.where(lane, cj, ali)
                    l = [jnp.where(pos[c] == cj, neg, l[c])
                         for c in range(_NV)]
                    return (*g, *l, atv, ati, alv, ali)

                zf = jnp.zeros((16,), jnp.float32)
                zi = jnp.zeros((16,), jnp.int32)
                carry = lax.fori_loop(0, 16, step, (*gv, *lvv, zf, zi, zf, zi))
                gv = list(carry[0:_NV])
                lvv = list(carry[_NV:2 * _NV])
                atv, ati, alv, ali = carry[2 * _NV:]
                o = h * BEAM + b * 16
                tv_v[pl.ds(o, 16)] = atv
                ti_v[pl.ds(o, 16)] = ati
                lv_v[pl.ds(o, 16)] = alv
                li_v[pl.ds(o, 16)] = ali

        obase = wid * (_ROWS_W * BEAM)
        pltpu.sync_copy(tv_v, tv_hbm.at[pl.ds(obase, _ROWS_W * BEAM)])
        pltpu.sync_copy(ti_v, ti_hbm.at[pl.ds(obase, _ROWS_W * BEAM)])
        pltpu.sync_copy(lv_v, lv_hbm.at[pl.ds(obase, _ROWS_W * BEAM)])
        pltpu.sync_copy(li_v, li_hbm.at[pl.ds(obase, _ROWS_W * BEAM)])

    return body(scores, part_flat)


def kernel(weighted_scores, part_ids):
    tv, ti, lv, li = _sc_beam(weighted_scores, part_ids.reshape(-1))
    return (tv.reshape(BATCH, BEAM), ti.reshape(BATCH, BEAM),
            lv.reshape(BATCH, BEAM), li.reshape(BATCH, BEAM))


# speedup vs baseline: 1.1325x; 1.0342x over previous
import functools

import jax
import jax.numpy as jnp
from jax import lax
from jax.experimental import pallas as pl
from jax.experimental.pallas import tpu as pltpu
from jax.experimental.pallas import tpu_sc as plsc

BEAM = 64
BATCH = 64
PRE = 96
VOCAB = 1000000

_NC = 2
_NS = 16
_NW = _NC * _NS
_PER_W = (BATCH * PRE) // _NW
_ROWS_W = _PER_W // PRE
_SEG = 128
_NV = PRE // 16


def _sc_beam(scores, part_flat):
    mesh = plsc.VectorSubcoreMesh(core_axis_name="c", subcore_axis_name="s")
    out_types = [
        jax.ShapeDtypeStruct((BATCH * BEAM,), jnp.float32),
        jax.ShapeDtypeStruct((BATCH * BEAM,), jnp.int32),
        jax.ShapeDtypeStruct((BATCH * BEAM,), jnp.float32),
        jax.ShapeDtypeStruct((BATCH * BEAM,), jnp.int32),
    ]

    @functools.partial(
        pl.kernel,
        mesh=mesh,
        out_type=out_types,
        scratch_types=[
            pltpu.VMEM((_PER_W,), jnp.int32),
            pltpu.VMEM((4, 16, 8, _SEG), jnp.float32),
            pltpu.VMEM((_PER_W,), jnp.float32),
            pltpu.VMEM((_ROWS_W * BEAM,), jnp.float32),
            pltpu.VMEM((_ROWS_W * BEAM,), jnp.int32),
            pltpu.VMEM((_ROWS_W * BEAM,), jnp.float32),
            pltpu.VMEM((_ROWS_W * BEAM,), jnp.int32),
            pltpu.SemaphoreType.DMA,
            pltpu.SemaphoreType.DMA,
            pltpu.SemaphoreType.DMA,
            pltpu.SemaphoreType.DMA,
        ],
        compiler_params=pltpu.CompilerParams(use_tc_tiling_on_sc=True,
                                             needs_layout_passes=False),
    )
    def body(scores_hbm, part_hbm, tv_hbm, ti_hbm, lv_hbm, li_hbm,
             part_v, bufs, vals_v, tv_v, ti_v, lv_v, li_v,
             sem_a, sem_b, sem_c, sem_d):
        wid = lax.axis_index("s") * _NC + lax.axis_index("c")
        base = wid * _PER_W
        pltpu.sync_copy(part_hbm.at[pl.ds(base, _PER_W)], part_v)
        iota16 = lax.iota(jnp.int32, 16)
        neg = jnp.float32(-jnp.inf)
        big = jnp.int32(2**30)
        sems = (sem_a, sem_b, sem_c, sem_d)
        depth = len(sems)

        n_groups = _ROWS_W * (PRE // 16)

        def issue(gl):
            h, g = divmod(gl, PRE // 16)
            row = wid * _ROWS_W + h
            row8 = pl.multiple_of((row // 8) * 8, 8)
            off = h * PRE + g * 16
            chunk = part_v[pl.ds(off, 16)]
            segs = (chunk // _SEG) * _SEG
            copies = []
            for i in range(16):
                col0 = pl.multiple_of(segs[i], _SEG)
                copies.append(pltpu.async_copy(
                    scores_hbm.at[pl.ds(row8, 8), pl.ds(col0, _SEG)],
                    bufs.at[gl % depth, i], sems[gl % depth]))
            return copies, chunk, off, row

        def drain(gl, state):
            copies, chunk, off, row = state
            for cp in copies:
                cp.wait()
            subl = jnp.full((16,), row & 7, jnp.int32)
            lanes = chunk & (_SEG - 1)
            vals_v[pl.ds(off, 16)] = plsc.load_gather(
                bufs.at[gl % depth], [iota16, subl, lanes])

        inflight = [issue(gl) for gl in range(depth - 1)]
        for gl in range(depth - 1, n_groups):
            inflight.append(issue(gl))
            drain(gl - depth + 1, inflight.pop(0))
        for k, st in enumerate(inflight):
            drain(n_groups - len(inflight) + k, st)

        for h in range(_ROWS_W):
            off = h * PRE
            gv = [vals_v[pl.ds(off + 16 * c, 16)] for c in range(_NV)]
            idv = [part_v[pl.ds(off + 16 * c, 16)] for c in range(_NV)]
            pos = [iota16 + 16 * c for c in range(_NV)]
            lvv = list(gv)
            for b in range(BEAM // 16):

                def step(i, carry):
                    g = list(carry[0:_NV])
                    l = list(carry[_NV:2 * _NV])
                    atv, ati, alv, ali = carry[2 * _NV:]
                    lane = iota16 == i
                    m = g[0]
                    for c in range(1, _NV):
                        m = jnp.maximum(m, g[c])
                    mx = jnp.max(m)
                    cidv = jnp.where(g[0] == mx, idv[0], big)
                    for c in range(1, _NV):
                        cidv = jnp.minimum(cidv,
                                           jnp.where(g[c] == mx, idv[c], big))
                    cid = jnp.min(cidv)
                    atv = jnp.where(lane, mx, atv)
                    ati = jnp.where(lane, cid, ati)
                    g = [jnp.where(idv[c] == cid, neg, g[c])
                         for c in range(_NV)]
                    ml = l[0]
                    for c in range(1, _NV):
                        ml = jnp.maximum(ml, l[c])
                    mlx = jnp.max(ml)
                    cjv = jnp.where(l[0] == mlx, pos[0], big)
                    for c in range(1, _NV):
                        cjv = jnp.minimum(cjv,
                                          jnp.where(l[c] == mlx, pos[c], big))
                    cj = jnp.min(cjv)
                    alv = jnp.where(lane, mlx, alv)
                    ali = jnp.where(lane, cj, ali)
                    l = [jnp.where(pos[c] == cj, neg, l[c])
                         for c in range(_NV)]
                    return (*g, *l, atv, ati, alv, ali)

                zf = jnp.zeros((16,), jnp.float32)
                zi = jnp.zeros((16,), jnp.int32)
                carry = lax.fori_loop(0, 16, step, (*gv, *lvv, zf, zi, zf, zi))
                gv = list(carry[0:_NV])
                lvv = list(carry[_NV:2 * _NV])
                atv, ati, alv, ali = carry[2 * _NV:]
                o = h * BEAM + b * 16
                tv_v[pl.ds(o, 16)] = atv
                ti_v[pl.ds(o, 16)] = ati
                lv_v[pl.ds(o, 16)] = alv
                li_v[pl.ds(o, 16)] = ali

        obase = wid * (_ROWS_W * BEAM)
        pltpu.sync_copy(tv_v, tv_hbm.at[pl.ds(obase, _ROWS_W * BEAM)])
        pltpu.sync_copy(ti_v, ti_hbm.at[pl.ds(obase, _ROWS_W * BEAM)])
        pltpu.sync_copy(lv_v, lv_hbm.at[pl.ds(obase, _ROWS_W * BEAM)])
        pltpu.sync_copy(li_v, li_hbm.at[pl.ds(obase, _ROWS_W * BEAM)])

    return body(scores, part_flat)


def kernel(weighted_scores, part_ids):
    tv, ti, lv, li = _sc_beam(weighted_scores, part_ids.reshape(-1))
    return (tv.reshape(BATCH, BEAM), ti.reshape(BATCH, BEAM),
            lv.reshape(BATCH, BEAM), li.reshape(BATCH, BEAM))


# speedup vs baseline: 1.4151x; 1.2495x over previous
import functools

import jax
import jax.numpy as jnp
from jax import lax
from jax.experimental import pallas as pl
from jax.experimental.pallas import tpu as pltpu
from jax.experimental.pallas import tpu_sc as plsc

BEAM = 64
BATCH = 64
PRE = 96
VOCAB = 1000000

_NC = 2
_NS = 16
_ROWS_W = 2
_ROWS_C = _NS * _ROWS_W
_SEG = 128
_NV = PRE // 16
_DEPTH = 6


def _sc_beam(scores, part_ids):
    mesh = plsc.VectorSubcoreMesh(core_axis_name="c", subcore_axis_name="s")
    out_types = [
        jax.ShapeDtypeStruct((BATCH, BEAM), jnp.float32),
        jax.ShapeDtypeStruct((BATCH, BEAM), jnp.int32),
        jax.ShapeDtypeStruct((BATCH, BEAM), jnp.float32),
        jax.ShapeDtypeStruct((BATCH, BEAM), jnp.int32),
    ]

    @functools.partial(
        pl.kernel,
        mesh=mesh,
        out_type=out_types,
        scratch_types=[
            pltpu.VMEM((8, PRE), jnp.int32),
            pltpu.VMEM((_DEPTH, 16, 8, _SEG), jnp.float32),
            pltpu.VMEM((_ROWS_W, PRE), jnp.float32),
            pltpu.VMEM((_ROWS_W, BEAM), jnp.float32),
            pltpu.VMEM((_ROWS_W, BEAM), jnp.int32),
            pltpu.VMEM((_ROWS_W, BEAM), jnp.float32),
            pltpu.VMEM((_ROWS_W, BEAM), jnp.int32),
            pltpu.VMEM_SHARED((_ROWS_C, BEAM), jnp.float32),
            pltpu.VMEM_SHARED((_ROWS_C, BEAM), jnp.int32),
            pltpu.VMEM_SHARED((_ROWS_C, BEAM), jnp.float32),
            pltpu.VMEM_SHARED((_ROWS_C, BEAM), jnp.int32),
        ] + [pltpu.SemaphoreType.DMA] * _DEPTH,
        compiler_params=pltpu.CompilerParams(use_tc_tiling_on_sc=True,
                                             needs_layout_passes=False),
    )
    def body(scores_hbm, part_hbm, tv_hbm, ti_hbm, lv_hbm, li_hbm,
             part_v, bufs, vals_v, tv_v, ti_v, lv_v, li_v,
             tv_sh, ti_sh, lv_sh, li_sh, *sems):
        cid_ax = lax.axis_index("c")
        sid = lax.axis_index("s")
        row0 = cid_ax * _ROWS_C + sid * _ROWS_W
        iota16 = lax.iota(jnp.int32, 16)
        neg = jnp.float32(-jnp.inf)
        big = jnp.int32(2**30)

        row8 = pl.multiple_of((row0 // 8) * 8, 8)
        pltpu.sync_copy(part_hbm.at[pl.ds(row8, 8), pl.ds(0, PRE)], part_v)

        n_groups = _ROWS_W * (PRE // 16)

        def issue(gl):
            h, g = divmod(gl, PRE // 16)
            row = row0 + h
            chunk = part_v[row & 7, pl.ds(g * 16, 16)]
            segs = (chunk // _SEG) * _SEG
            copies = []
            for i in range(16):
                col0 = pl.multiple_of(segs[i], _SEG)
                copies.append(pltpu.async_copy(
                    scores_hbm.at[pl.ds(row8, 8), pl.ds(col0, _SEG)],
                    bufs.at[gl % _DEPTH, i], sems[gl % _DEPTH]))
            return copies, chunk, row

        def drain(gl, state):
            copies, chunk, row = state
            for cp in copies:
                cp.wait()
            h, g = divmod(gl, PRE // 16)
            subl = jnp.full((16,), row & 7, jnp.int32)
            lanes = chunk & (_SEG - 1)
            vals_v[h, pl.ds(g * 16, 16)] = plsc.load_gather(
                bufs.at[gl % _DEPTH], [iota16, subl, lanes])

        def topk(h):
            row = row0 + h
            gv = [vals_v[h, pl.ds(16 * c, 16)] for c in range(_NV)]
            idv = [part_v[row & 7, pl.ds(16 * c, 16)] for c in range(_NV)]
            pos = [iota16 + 16 * c for c in range(_NV)]
            lvv = list(gv)
            for b in range(BEAM // 16):

                def step(i, carry):
                    g = list(carry[0:_NV])
                    l = list(carry[_NV:2 * _NV])
                    atv, ati, alv, ali = carry[2 * _NV:]
                    lane = iota16 == i
                    m = g[0]
                    for c in range(1, _NV):
                        m = jnp.maximum(m, g[c])
                    mx = jnp.max(m)
                    cv = jnp.where(g[0] == mx, idv[0], big)
                    for c in range(1, _NV):
                        cv = jnp.minimum(cv,
                                         jnp.where(g[c] == mx, idv[c], big))
                    cid = jnp.min(cv)
                    atv = jnp.where(lane, mx, atv)
                    ati = jnp.where(lane, cid, ati)
                    g = [jnp.where(idv[c] == cid, neg, g[c])
                         for c in range(_NV)]
                    ml = l[0]
                    for c in range(1, _NV):
                        ml = jnp.maximum(ml, l[c])
                    mlx = jnp.max(ml)
                    cjv = jnp.where(l[0] == mlx, pos[0], big)
                    for c in range(1, _NV):
                        cjv = jnp.minimum(cjv,
                                          jnp.where(l[c] == mlx, pos[c], big))
                    cj = jnp.min(cjv)
                    alv = jnp.where(lane, mlx, alv)
                    ali = jnp.where(lane, cj, ali)
                    l = [jnp.where(pos[c] == cj, neg, l[c])
                         for c in range(_NV)]
                    return (*g, *l, atv, ati, alv, ali)

                zf = jnp.zeros((16,), jnp.float32)
                zi = jnp.zeros((16,), jnp.int32)
                carry = lax.fori_loop(0, 16, step, (*gv, *lvv, zf, zi, zf, zi))
                gv = list(carry[0:_NV])
                lvv = list(carry[_NV:2 * _NV])
                atv, ati, alv, ali = carry[2 * _NV:]
                tv_v[h, pl.ds(b * 16, 16)] = atv
                ti_v[h, pl.ds(b * 16, 16)] = ati
                lv_v[h, pl.ds(b * 16, 16)] = alv
                li_v[h, pl.ds(b * 16, 16)] = ali

        inflight = [issue(gl) for gl in range(_DEPTH)]
        for gl in range(_DEPTH, n_groups):
            inflight.append(issue(gl))
            drain(gl - _DEPTH, inflight.pop(0))
        for k, st in enumerate(inflight):
            d = n_groups - len(inflight) + k
            drain(d, st)
            if d == PRE // 16 - 1:
                topk(0)
        topk(1)

        lrow = sid * _ROWS_W
        pltpu.sync_copy(tv_v, tv_sh.at[pl.ds(lrow, _ROWS_W)])
        pltpu.sync_copy(ti_v, ti_sh.at[pl.ds(lrow, _ROWS_W)])
        pltpu.sync_copy(lv_v, lv_sh.at[pl.ds(lrow, _ROWS_W)])
        pltpu.sync_copy(li_v, li_sh.at[pl.ds(lrow, _ROWS_W)])
        plsc.subcore_barrier()

        outs = (tv_hbm, ti_hbm, lv_hbm, li_hbm)
        shs = (tv_sh, ti_sh, lv_sh, li_sh)
        t = sid % 4
        off0 = pl.multiple_of(cid_ax * _ROWS_C + t * 8, 8)
        for oi in range(4):

            @pl.when(sid // 4 == oi)
            def _(oi=oi):
                pltpu.sync_copy(
                    shs[oi].at[pl.ds(t * 8, 8)],
                    outs[oi].at[pl.ds(off0, 8), pl.ds(0, BEAM)])

    return body(scores, part_ids)


def kernel(weighted_scores, part_ids):
    tv, ti, lv, li = _sc_beam(weighted_scores, part_ids)
    return (tv, ti, lv, li)
